# baseline (speedup 1.0000x reference)
import jax
import jax.numpy as jnp
from jax.experimental import pallas as pl


def _map_body(map_t, map_pe, pos, map_o):
    S = map_t.shape[1]
    map_o[...] = map_t[...] + (map_pe[...] + pos[:S])[None]


def _actor_body(actor_t, actor_pe, time_pe, pos, actor_o):
    T = actor_t.shape[1]
    N = actor_t.shape[2]
    D = actor_t.shape[-1]
    pos_all = pos[...]
    time_comb = (time_pe[:T] + pos_all[:T]).reshape(1, T, 1, D)
    actor_comb = (actor_pe[:N] + pos_all[:N]).reshape(1, 1, N, D)
    actor_o[...] = actor_t[...] + actor_comb + time_comb


def _light_body(light_t, light_pe, time_pe, pos, light_o):
    T = light_t.shape[1]
    L = light_t.shape[2]
    D = light_t.shape[-1]
    pos_all = pos[...]
    time_comb = (time_pe[:T] + pos_all[:T]).reshape(1, T, 1, D)
    light_comb = (light_pe[:L] + pos_all[:L]).reshape(1, 1, L, D)
    light_o[...] = light_t[...] + light_comb + time_comb


def _stream(body, token, tables, g):
    B = token.shape[0]
    blk = (g,) + token.shape[1:]
    nd = len(blk)
    whole = lambda shape: pl.BlockSpec(shape, lambda b: (0,) * len(shape))
    return pl.pallas_call(
        body,
        grid=(B // g,),
        in_specs=[pl.BlockSpec(blk, lambda b: (b,) + (0,) * (nd - 1))]
        + [whole(t.shape) for t in tables],
        out_specs=pl.BlockSpec(blk, lambda b: (b,) + (0,) * (nd - 1)),
        out_shape=jax.ShapeDtypeStruct(token.shape, token.dtype),
    )(token, *tables)


def kernel(map_token, actor_token, light_token, map_pe_w, actor_pe_w,
           light_pe_w, time_pe_w, pos_enc):
    B = map_token.shape[0]
    g8 = 8 if B % 8 == 0 else 1
    g2 = 2 if B % 2 == 0 else 1
    map_o = _stream(_map_body, map_token, (map_pe_w, pos_enc), g8)
    actor_o = _stream(_actor_body, actor_token,
                      (actor_pe_w, time_pe_w, pos_enc), g2)
    light_o = _stream(_light_body, light_token,
                      (light_pe_w, time_pe_w, pos_enc), g8)
    return (map_o, actor_o, light_o)
